# baseline (speedup 1.0000x reference)
import functools

import jax
import jax.numpy as jnp
from jax import lax
from jax.experimental import pallas as pl
from jax.experimental.pallas import tpu as pltpu
from jax.experimental.pallas import tpu_sc as plsc

N = 10000
D = 128
EDGE_DIM = 16
E = 320000
N_LAYERS = 3

NW = 32
CHUNK = 128
E8 = E // 8
GPW = 40960
E_PAD = 8 * GPW
HALF = E_PAD // 2
NC_H = HALF // NW // CHUNK
N_PAD = 10240

NB = 400
EB = 640
NI = GPW // EB


@functools.cache
def _sc_gather_kernel():
    mesh = plsc.VectorSubcoreMesh(core_axis_name="c", subcore_axis_name="s")

    @functools.partial(
        pl.kernel,
        mesh=mesh,
        out_type=jax.ShapeDtypeStruct((NW, NC_H, CHUNK, D), jnp.float32),
        scratch_types=[
            pltpu.VMEM((NC_H, CHUNK), jnp.int32),
            pltpu.VMEM((CHUNK, D), jnp.float32),
            pltpu.VMEM((CHUNK, D), jnp.float32),
            pltpu.VMEM_SHARED((N, D), jnp.float32),
            pltpu.SemaphoreType.DMA,
            pltpu.SemaphoreType.DMA,
        ],
    )
    def gather_k(table_hbm, idx_hbm, out_hbm, idx_v, buf0, buf1, tbl,
                 sem0, sem1):
        c = lax.axis_index("c")
        s = lax.axis_index("s")
        wid = s * 2 + c

        pltpu.sync_copy(table_hbm.at[pl.ds(s * 624, 624)],
                        tbl.at[pl.ds(s * 624, 624)])

        @pl.when(s == 0)
        def _():
            pltpu.sync_copy(table_hbm.at[pl.ds(9984, 16)],
                            tbl.at[pl.ds(9984, 16)])

        pltpu.sync_copy(idx_hbm.at[wid], idx_v)
        plsc.subcore_barrier()

        def body(i, carry):
            j0 = 2 * i
            j1 = j0 + 1
            c0 = pltpu.async_copy(tbl.at[idx_v.at[j0]], buf0, sem0)
            c1 = pltpu.async_copy(tbl.at[idx_v.at[j1]], buf1, sem1)
            c0.wait()
            pltpu.sync_copy(buf0, out_hbm.at[wid, j0])
            c1.wait()
            pltpu.sync_copy(buf1, out_hbm.at[wid, j1])
            return carry

        lax.fori_loop(0, NC_H // 2, body, 0)

    return gather_k


def _sc_gather(table, idx_r):
    return _sc_gather_kernel()(table, idx_r)


@functools.cache
def _sc_scatter_kernel():
    mesh = plsc.VectorSubcoreMesh(core_axis_name="c", subcore_axis_name="s")

    @functools.partial(
        pl.kernel,
        mesh=mesh,
        out_type=jax.ShapeDtypeStruct((2, N_PAD, D), jnp.float32),
        scratch_types=[
            pltpu.VMEM((NC_H, CHUNK), jnp.int32),
            pltpu.VMEM((CHUNK, D), jnp.float32),
            pltpu.VMEM((CHUNK, D), jnp.float32),
            pltpu.VMEM_SHARED((N_PAD, D), jnp.float32),
            pltpu.SemaphoreType.DMA,
            pltpu.SemaphoreType.DMA,
        ],
    )
    def scatter_k(m_hbm, idx_hbm, z_hbm, out_hbm,
                  idx_v, buf0, buf1, acc, sem0, sem1):
        c = lax.axis_index("c")
        s = lax.axis_index("s")
        wid = s * 2 + c
        rows_per_sub = N_PAD // 16

        pltpu.sync_copy(z_hbm, buf0)

        def zbody(t, carry):
            pltpu.sync_copy(
                buf0, acc.at[pl.ds(s * rows_per_sub + t * CHUNK, CHUNK)])
            return carry

        lax.fori_loop(0, rows_per_sub // CHUNK, zbody, 0)
        pltpu.sync_copy(idx_hbm.at[wid], idx_v)
        plsc.subcore_barrier()

        def body(i, carry):
            j0 = 2 * i
            j1 = j0 + 1
            c0 = pltpu.async_copy(m_hbm.at[wid, j0], buf0, sem0)
            c1 = pltpu.async_copy(m_hbm.at[wid, j1], buf1, sem1)
            c0.wait()
            pltpu.sync_copy(buf0, acc.at[idx_v.at[j0]], add=True)
            c1.wait()
            pltpu.sync_copy(buf1, acc.at[idx_v.at[j1]], add=True)
            return carry

        lax.fori_loop(0, NC_H // 2, body, 0)
        plsc.subcore_barrier()

        pltpu.sync_copy(acc.at[pl.ds(s * rows_per_sub, rows_per_sub)],
                        out_hbm.at[c, pl.ds(s * rows_per_sub, rows_per_sub)])

    return scatter_k


def _sc_scatter(m_r, idx_r, zeros_blk):
    return _sc_scatter_kernel()(m_r, idx_r, zeros_blk)


def _full(shape):
    return pl.BlockSpec(shape, lambda i: (0,) * len(shape))


def _full2(shape):
    return pl.BlockSpec(shape, lambda i, c: (0,) * len(shape))


def _node_tc(x, mp, ln_g, ln_b, w1h_msg):
    def body(x_ref, w1, b1, w2, b2, w3, b3, g, b, wm, h_ref, p_ref):
        xv = jnp.nan_to_num(x_ref[...], nan=0.0, posinf=0.0, neginf=0.0)
        h = jnp.maximum(xv @ w1[...] + b1[...], 0.0)
        h = jnp.maximum(h @ w2[...] + b2[...], 0.0)
        h = h @ w3[...] + b3[...]
        mu = jnp.mean(h, axis=-1, keepdims=True)
        var = jnp.mean((h - mu) ** 2, axis=-1, keepdims=True)
        hn = (h - mu) * lax.rsqrt(var + 1e-5) * g[...] + b[...]
        h_ref[...] = hn
        p_ref[...] = hn @ wm[...]

    return pl.pallas_call(
        body,
        grid=(N // NB,),
        in_specs=[
            pl.BlockSpec((NB, D), lambda i: (i, 0)),
            _full((D, D)), _full((1, D)), _full((D, D)), _full((1, D)),
            _full((D, D)), _full((1, D)), _full((1, D)), _full((1, D)),
            _full((D, D)),
        ],
        out_specs=[
            pl.BlockSpec((NB, D), lambda i: (i, 0)),
            pl.BlockSpec((NB, D), lambda i: (i, 0)),
        ],
        out_shape=[
            jax.ShapeDtypeStruct((N, D), jnp.float32),
            jax.ShapeDtypeStruct((N, D), jnp.float32),
        ],
    )(x, mp['W1'], mp['b1'].reshape(1, D), mp['W2'], mp['b2'].reshape(1, D),
      mp['W3'], mp['b3'].reshape(1, D), ln_g.reshape(1, D), ln_b.reshape(1, D),
      w1h_msg)


def _msg_tc(g_arr, ea2p, wbig_h, b1, w2, b2, w3, b3):
    def mm(a, w):
        return jax.lax.dot(a.astype(jnp.bfloat16), w.astype(jnp.bfloat16),
                           preferred_element_type=jnp.float32)

    def body(g_ref, ea_ref, wb_ref, b1r, w2r, b2r, w3r, b3r, m_ref):
        c = pl.program_id(1)
        ea = jnp.nan_to_num(ea_ref[...], nan=0.0, posinf=0.0, neginf=0.0)
        wc = wb_ref[c]
        h = jnp.maximum(g_ref[...] + mm(ea, wc) + b1r[...], 0.0)
        h = jnp.maximum(mm(h, w2r[...]) + b2r[...], 0.0)
        m_ref[...] = mm(h, w3r[...]) + b3r[...]

    return pl.pallas_call(
        body,
        grid=(NI, 4),
        in_specs=[
            pl.BlockSpec((EB, D), lambda i, c: (c * NI + i, 0)),
            pl.BlockSpec((EB, D), lambda i, c: (i, 0)),
            pl.BlockSpec((4, D, D), lambda i, c: (0, 0, 0)),
            _full2((1, D)), _full2((D, D)), _full2((1, D)),
            _full2((D, D)), _full2((1, D)),
        ],
        out_specs=pl.BlockSpec((EB, D), lambda i, c: (c * NI + i, 0)),
        out_shape=jax.ShapeDtypeStruct((HALF, D), jnp.float32),
    )(g_arr, ea2p, wbig_h, b1.reshape(1, D), w2, b2.reshape(1, D), w3,
      b3.reshape(1, D))


def _update_tc(h, part_a, part_b, up, ln_g, ln_b, w1h_msg, compute_mean):
    nb = N // NB
    w1 = up['W1']

    def body(h_ref, pa0_ref, pa1_ref, pb0_ref, pb1_ref,
             w1h, w1a, b1, w2, b2, w3, b3, g, b, wm, *outs):
        agg = (pa0_ref[0] + pa1_ref[0]) + (pb0_ref[0] + pb1_ref[0])
        hv = h_ref[...]
        u = jnp.maximum(hv @ w1h[...] + agg @ w1a[...] + b1[...], 0.0)
        u = jnp.maximum(u @ w2[...] + b2[...], 0.0)
        u = u @ w3[...] + b3[...]
        hh = hv + u
        mu = jnp.mean(hh, axis=-1, keepdims=True)
        var = jnp.mean((hh - mu) ** 2, axis=-1, keepdims=True)
        hn = (hh - mu) * lax.rsqrt(var + 1e-5) * g[...] + b[...]
        outs[0][...] = hn
        outs[1][...] = hn @ wm[...]
        if compute_mean:
            i = pl.program_id(0)
            gacc = outs[2]

            @pl.when(i == 0)
            def _():
                gacc[...] = jnp.zeros_like(gacc)

            gacc[...] += jnp.sum(hn, axis=0, keepdims=True)

            @pl.when(i == nb - 1)
            def _():
                gacc[...] = gacc[...] * (1.0 / N)

    out_specs = [
        pl.BlockSpec((NB, D), lambda i: (i, 0)),
        pl.BlockSpec((NB, D), lambda i: (i, 0)),
    ]
    out_shape = [
        jax.ShapeDtypeStruct((N, D), jnp.float32),
        jax.ShapeDtypeStruct((N, D), jnp.float32),
    ]
    if compute_mean:
        out_specs.append(_full((1, D)))
        out_shape.append(jax.ShapeDtypeStruct((1, D), jnp.float32))

    return pl.pallas_call(
        body,
        grid=(nb,),
        in_specs=[
            pl.BlockSpec((NB, D), lambda i: (i, 0)),
            pl.BlockSpec((1, NB, D), lambda i: (0, i, 0)),
            pl.BlockSpec((1, NB, D), lambda i: (1, i, 0)),
            pl.BlockSpec((1, NB, D), lambda i: (0, i, 0)),
            pl.BlockSpec((1, NB, D), lambda i: (1, i, 0)),
            _full((D, D)), _full((D, D)), _full((1, D)), _full((D, D)),
            _full((1, D)), _full((D, D)), _full((1, D)), _full((1, D)),
            _full((1, D)), _full((D, D)),
        ],
        out_specs=out_specs,
        out_shape=out_shape,
    )(h, part_a, part_a, part_b, part_b, w1[:D], w1[D:],
      up['b1'].reshape(1, D), up['W2'],
      up['b2'].reshape(1, D), up['W3'], up['b3'].reshape(1, D),
      ln_g.reshape(1, D), ln_b.reshape(1, D), w1h_msg)


def kernel(node_x, edge_index, edge_attr, params):
    node_x = node_x.astype(jnp.float32)
    edge_attr = edge_attr.astype(jnp.float32)
    src = edge_index[0].astype(jnp.int32)
    dst = edge_index[1].astype(jnp.int32)

    srcg = jnp.pad(src.reshape(E8, 8).T, ((0, 0), (0, GPW - E8)))
    dstg = jnp.pad(dst.reshape(E8, 8).T, ((0, 0), (0, GPW - E8)),
                   constant_values=N)
    src_h = srcg.reshape(2, NW, NC_H, CHUNK)
    dst_h = dstg.reshape(2, NW, NC_H, CHUNK)
    ea2p = jnp.pad(edge_attr.reshape(E8, 8 * EDGE_DIM),
                   ((0, GPW - E8), (0, 0)))

    mp = params['msg_mlp']
    w1h_msg = mp['W1'][:D]
    w1e = mp['W1'][D:]
    wbig = jnp.zeros((8, D, D), jnp.float32)
    for c in range(8):
        wbig = wbig.at[c, 16 * c:16 * (c + 1), :].set(w1e)
    ln_g, ln_b = params['ln_g'], params['ln_b']

    H, P = _node_tc(node_x, params['node_mlp'], ln_g, ln_b, w1h_msg)
    zeros_blk = jnp.zeros((CHUNK, D), jnp.float32)

    gsum = None
    for layer in range(N_LAYERS):
        parts = []
        for half in range(2):
            G = _sc_gather(P, src_h[half]).reshape(HALF, D)
            M = _msg_tc(G, ea2p, wbig[4 * half:4 * half + 4], mp['b1'],
                        mp['W2'], mp['b2'], mp['W3'], mp['b3'])
            parts.append(_sc_scatter(M.reshape(NW, NC_H, CHUNK, D),
                                     dst_h[half], zeros_blk))
        last = layer == N_LAYERS - 1
        if last:
            H, P, gsum = _update_tc(H, parts[0], parts[1], params['up_mlp'],
                                    ln_g, ln_b, w1h_msg, True)
        else:
            H, P = _update_tc(H, parts[0], parts[1], params['up_mlp'],
                              ln_g, ln_b, w1h_msg, False)

    return (H, gsum.reshape(D))


# speedup vs baseline: 1.0431x; 1.0431x over previous
import functools

import jax
import jax.numpy as jnp
from jax import lax
from jax.experimental import pallas as pl
from jax.experimental.pallas import tpu as pltpu
from jax.experimental.pallas import tpu_sc as plsc

N = 10000
D = 128
EDGE_DIM = 16
E = 320000
N_LAYERS = 3

NW = 32
CHUNK = 128
E8 = E // 8
GPW = 40960
E_PAD = 8 * GPW
HALF = E_PAD // 2
NC_H = HALF // NW // CHUNK
N_PAD = 10240

NB = 400
EB = 640
NI = GPW // EB


@functools.cache
def _sc_gather_kernel():
    mesh = plsc.VectorSubcoreMesh(core_axis_name="c", subcore_axis_name="s")

    @functools.partial(
        pl.kernel,
        mesh=mesh,
        out_type=jax.ShapeDtypeStruct((NW, NC_H, CHUNK, D), jnp.float32),
        scratch_types=[
            pltpu.VMEM((NC_H, CHUNK), jnp.int32),
            pltpu.VMEM((CHUNK, D), jnp.float32),
            pltpu.VMEM((CHUNK, D), jnp.float32),
            pltpu.VMEM_SHARED((N, D), jnp.float32),
            pltpu.SemaphoreType.DMA,
            pltpu.SemaphoreType.DMA,
        ],
    )
    def gather_k(table_hbm, idx_hbm, out_hbm, idx_v, buf0, buf1, tbl,
                 sem0, sem1):
        c = lax.axis_index("c")
        s = lax.axis_index("s")
        wid = s * 2 + c

        pltpu.sync_copy(table_hbm.at[pl.ds(s * 624, 624)],
                        tbl.at[pl.ds(s * 624, 624)])

        @pl.when(s == 0)
        def _():
            pltpu.sync_copy(table_hbm.at[pl.ds(9984, 16)],
                            tbl.at[pl.ds(9984, 16)])

        pltpu.sync_copy(idx_hbm.at[wid], idx_v)
        plsc.subcore_barrier()

        def body(i, carry):
            j0 = 2 * i
            j1 = j0 + 1
            c0 = pltpu.async_copy(tbl.at[idx_v.at[j0]], buf0, sem0)
            c1 = pltpu.async_copy(tbl.at[idx_v.at[j1]], buf1, sem1)
            c0.wait()
            pltpu.sync_copy(buf0, out_hbm.at[wid, j0])
            c1.wait()
            pltpu.sync_copy(buf1, out_hbm.at[wid, j1])
            return carry

        lax.fori_loop(0, NC_H // 2, body, 0)

    return gather_k


def _sc_gather(table, idx_r):
    return _sc_gather_kernel()(table, idx_r)


@functools.cache
def _sc_scatter_kernel():
    mesh = plsc.VectorSubcoreMesh(core_axis_name="c", subcore_axis_name="s")

    @functools.partial(
        pl.kernel,
        mesh=mesh,
        out_type=jax.ShapeDtypeStruct((2, N_PAD, D), jnp.float32),
        scratch_types=[
            pltpu.VMEM((NC_H, CHUNK), jnp.int32),
            pltpu.VMEM((CHUNK, D), jnp.float32),
            pltpu.VMEM((CHUNK, D), jnp.float32),
            pltpu.VMEM_SHARED((N_PAD, D), jnp.float32),
            pltpu.SemaphoreType.DMA,
            pltpu.SemaphoreType.DMA,
        ],
    )
    def scatter_k(m_hbm, idx_hbm, z_hbm, out_hbm,
                  idx_v, buf0, buf1, acc, sem0, sem1):
        c = lax.axis_index("c")
        s = lax.axis_index("s")
        wid = s * 2 + c
        rows_per_sub = N_PAD // 16

        pltpu.sync_copy(z_hbm, buf0)

        def zbody(t, carry):
            pltpu.sync_copy(
                buf0, acc.at[pl.ds(s * rows_per_sub + t * CHUNK, CHUNK)])
            return carry

        lax.fori_loop(0, rows_per_sub // CHUNK, zbody, 0)
        pltpu.sync_copy(idx_hbm.at[wid], idx_v)
        plsc.subcore_barrier()

        def body(i, carry):
            j0 = 2 * i
            j1 = j0 + 1
            c0 = pltpu.async_copy(m_hbm.at[wid, j0], buf0, sem0)
            c1 = pltpu.async_copy(m_hbm.at[wid, j1], buf1, sem1)
            c0.wait()
            pltpu.sync_copy(buf0, acc.at[idx_v.at[j0]], add=True)
            c1.wait()
            pltpu.sync_copy(buf1, acc.at[idx_v.at[j1]], add=True)
            return carry

        lax.fori_loop(0, NC_H // 2, body, 0)
        plsc.subcore_barrier()

        pltpu.sync_copy(acc.at[pl.ds(s * rows_per_sub, rows_per_sub)],
                        out_hbm.at[c, pl.ds(s * rows_per_sub, rows_per_sub)])

    return scatter_k


def _sc_scatter(m_r, idx_r, zeros_blk):
    return _sc_scatter_kernel()(m_r, idx_r, zeros_blk)


NCF = E_PAD // NW // CHUNK


@functools.cache
def _sc_deg_kernel():
    mesh = plsc.VectorSubcoreMesh(core_axis_name="c", subcore_axis_name="s")

    @functools.partial(
        pl.kernel,
        mesh=mesh,
        out_type=jax.ShapeDtypeStruct((2, N_PAD, D), jnp.float32),
        scratch_types=[
            pltpu.VMEM((NCF, CHUNK), jnp.int32),
            pltpu.VMEM((CHUNK, D), jnp.float32),
            pltpu.VMEM_SHARED((N_PAD, D), jnp.float32),
        ],
    )
    def deg_k(idx_hbm, z_hbm, ones_hbm, out_hbm, idx_v, buf0, acc):
        c = lax.axis_index("c")
        s = lax.axis_index("s")
        wid = s * 2 + c
        rows_per_sub = N_PAD // 16

        pltpu.sync_copy(z_hbm, buf0)

        def zbody(t, carry):
            pltpu.sync_copy(
                buf0, acc.at[pl.ds(s * rows_per_sub + t * CHUNK, CHUNK)])
            return carry

        lax.fori_loop(0, rows_per_sub // CHUNK, zbody, 0)
        pltpu.sync_copy(ones_hbm, buf0)
        pltpu.sync_copy(idx_hbm.at[wid], idx_v)
        plsc.subcore_barrier()

        def body(j, carry):
            pltpu.sync_copy(buf0, acc.at[idx_v.at[j]], add=True)
            return carry

        lax.fori_loop(0, NCF, body, 0)
        plsc.subcore_barrier()

        pltpu.sync_copy(acc.at[pl.ds(s * rows_per_sub, rows_per_sub)],
                        out_hbm.at[c, pl.ds(s * rows_per_sub, rows_per_sub)])

    return deg_k


def _sc_deg(idx_full, zeros_blk, ones_blk):
    return _sc_deg_kernel()(idx_full, zeros_blk, ones_blk)


def _full(shape):
    return pl.BlockSpec(shape, lambda i: (0,) * len(shape))


def _full2(shape):
    return pl.BlockSpec(shape, lambda i, c: (0,) * len(shape))


def _node_tc(x, mp, ln_g, ln_b, w1h_msg):
    def body(x_ref, w1, b1, w2, b2, w3, b3, g, b, wm, h_ref, p_ref):
        xv = jnp.nan_to_num(x_ref[...], nan=0.0, posinf=0.0, neginf=0.0)
        h = jnp.maximum(xv @ w1[...] + b1[...], 0.0)
        h = jnp.maximum(h @ w2[...] + b2[...], 0.0)
        h = h @ w3[...] + b3[...]
        mu = jnp.mean(h, axis=-1, keepdims=True)
        var = jnp.mean((h - mu) ** 2, axis=-1, keepdims=True)
        hn = (h - mu) * lax.rsqrt(var + 1e-5) * g[...] + b[...]
        h_ref[...] = hn
        p_ref[...] = hn @ wm[...]

    return pl.pallas_call(
        body,
        grid=(N // NB,),
        in_specs=[
            pl.BlockSpec((NB, D), lambda i: (i, 0)),
            _full((D, D)), _full((1, D)), _full((D, D)), _full((1, D)),
            _full((D, D)), _full((1, D)), _full((1, D)), _full((1, D)),
            _full((D, D)),
        ],
        out_specs=[
            pl.BlockSpec((NB, D), lambda i: (i, 0)),
            pl.BlockSpec((NB, D), lambda i: (i, 0)),
        ],
        out_shape=[
            jax.ShapeDtypeStruct((N, D), jnp.float32),
            jax.ShapeDtypeStruct((N, D), jnp.float32),
        ],
    )(x, mp['W1'], mp['b1'].reshape(1, D), mp['W2'], mp['b2'].reshape(1, D),
      mp['W3'], mp['b3'].reshape(1, D), ln_g.reshape(1, D), ln_b.reshape(1, D),
      w1h_msg)


def _msg_tc(g_arr, ea2p, wbig_h, b1, w2, b2):
    def body(g_ref, ea_ref, wb_ref, b1r, w2r, b2r, m_ref):
        c = pl.program_id(1)
        wc = wb_ref[c]
        h = jnp.maximum(g_ref[...] + ea_ref[...] @ wc + b1r[...], 0.0)
        m_ref[...] = jnp.maximum(h @ w2r[...] + b2r[...], 0.0)

    return pl.pallas_call(
        body,
        grid=(NI, 4),
        in_specs=[
            pl.BlockSpec((EB, D), lambda i, c: (c * NI + i, 0)),
            pl.BlockSpec((EB, D), lambda i, c: (i, 0)),
            pl.BlockSpec((4, D, D), lambda i, c: (0, 0, 0)),
            _full2((1, D)), _full2((D, D)), _full2((1, D)),
        ],
        out_specs=pl.BlockSpec((EB, D), lambda i, c: (c * NI + i, 0)),
        out_shape=jax.ShapeDtypeStruct((HALF, D), jnp.float32),
    )(g_arr, ea2p, wbig_h, b1.reshape(1, D), w2, b2.reshape(1, D))


def _update_tc(h, part_a, part_b, deg, w3m, b3m, up, ln_g, ln_b, w1h_msg,
               compute_mean):
    nb = N // NB
    w1 = up['W1']

    def body(h_ref, pa0_ref, pa1_ref, pb0_ref, pb1_ref, d0_ref, d1_ref,
             w3r, b3r, w1h, w1a, b1, w2, b2, w3, b3, g, b, wm, *outs):
        agg2 = (pa0_ref[0] + pa1_ref[0]) + (pb0_ref[0] + pb1_ref[0])
        d = (d0_ref[0] + d1_ref[0])[:, 0:1]
        agg = agg2 @ w3r[...] + d * b3r[...]
        hv = h_ref[...]
        u = jnp.maximum(hv @ w1h[...] + agg @ w1a[...] + b1[...], 0.0)
        u = jnp.maximum(u @ w2[...] + b2[...], 0.0)
        u = u @ w3[...] + b3[...]
        hh = hv + u
        mu = jnp.mean(hh, axis=-1, keepdims=True)
        var = jnp.mean((hh - mu) ** 2, axis=-1, keepdims=True)
        hn = (hh - mu) * lax.rsqrt(var + 1e-5) * g[...] + b[...]
        outs[0][...] = hn
        outs[1][...] = hn @ wm[...]
        if compute_mean:
            i = pl.program_id(0)
            gacc = outs[2]

            @pl.when(i == 0)
            def _():
                gacc[...] = jnp.zeros_like(gacc)

            gacc[...] += jnp.sum(hn, axis=0, keepdims=True)

            @pl.when(i == nb - 1)
            def _():
                gacc[...] = gacc[...] * (1.0 / N)

    out_specs = [
        pl.BlockSpec((NB, D), lambda i: (i, 0)),
        pl.BlockSpec((NB, D), lambda i: (i, 0)),
    ]
    out_shape = [
        jax.ShapeDtypeStruct((N, D), jnp.float32),
        jax.ShapeDtypeStruct((N, D), jnp.float32),
    ]
    if compute_mean:
        out_specs.append(_full((1, D)))
        out_shape.append(jax.ShapeDtypeStruct((1, D), jnp.float32))

    return pl.pallas_call(
        body,
        grid=(nb,),
        in_specs=[
            pl.BlockSpec((NB, D), lambda i: (i, 0)),
            pl.BlockSpec((1, NB, D), lambda i: (0, i, 0)),
            pl.BlockSpec((1, NB, D), lambda i: (1, i, 0)),
            pl.BlockSpec((1, NB, D), lambda i: (0, i, 0)),
            pl.BlockSpec((1, NB, D), lambda i: (1, i, 0)),
            pl.BlockSpec((1, NB, D), lambda i: (0, i, 0)),
            pl.BlockSpec((1, NB, D), lambda i: (1, i, 0)),
            _full((D, D)), _full((1, D)),
            _full((D, D)), _full((D, D)), _full((1, D)), _full((D, D)),
            _full((1, D)), _full((D, D)), _full((1, D)), _full((1, D)),
            _full((1, D)), _full((D, D)),
        ],
        out_specs=out_specs,
        out_shape=out_shape,
    )(h, part_a, part_a, part_b, part_b, deg, deg, w3m, b3m.reshape(1, D),
      w1[:D], w1[D:],
      up['b1'].reshape(1, D), up['W2'],
      up['b2'].reshape(1, D), up['W3'], up['b3'].reshape(1, D),
      ln_g.reshape(1, D), ln_b.reshape(1, D), w1h_msg)


def kernel(node_x, edge_index, edge_attr, params):
    node_x = node_x.astype(jnp.float32)
    edge_attr = edge_attr.astype(jnp.float32)
    src = edge_index[0].astype(jnp.int32)
    dst = edge_index[1].astype(jnp.int32)

    srcg = jnp.pad(src.reshape(E8, 8).T, ((0, 0), (0, GPW - E8)))
    dstg = jnp.pad(dst.reshape(E8, 8).T, ((0, 0), (0, GPW - E8)),
                   constant_values=N)
    src_h = srcg.reshape(2, NW, NC_H, CHUNK)
    dst_h = dstg.reshape(2, NW, NC_H, CHUNK)
    ea2p = jnp.pad(edge_attr.reshape(E8, 8 * EDGE_DIM),
                   ((0, GPW - E8), (0, 0)))

    mp = params['msg_mlp']
    w1h_msg = mp['W1'][:D]
    w1e = mp['W1'][D:]
    wbig = jnp.zeros((8, D, D), jnp.float32)
    for c in range(8):
        wbig = wbig.at[c, 16 * c:16 * (c + 1), :].set(w1e)
    ln_g, ln_b = params['ln_g'], params['ln_b']

    H, P = _node_tc(node_x, params['node_mlp'], ln_g, ln_b, w1h_msg)
    zeros_blk = jnp.zeros((CHUNK, D), jnp.float32)
    ones_blk = jnp.ones((CHUNK, D), jnp.float32)
    deg = _sc_deg(dstg.reshape(NW, NCF, CHUNK), zeros_blk, ones_blk)

    gsum = None
    for layer in range(N_LAYERS):
        parts = []
        for half in range(2):
            G = _sc_gather(P, src_h[half]).reshape(HALF, D)
            M = _msg_tc(G, ea2p, wbig[4 * half:4 * half + 4], mp['b1'],
                        mp['W2'], mp['b2'])
            parts.append(_sc_scatter(M.reshape(NW, NC_H, CHUNK, D),
                                     dst_h[half], zeros_blk))
        last = layer == N_LAYERS - 1
        if last:
            H, P, gsum = _update_tc(H, parts[0], parts[1], deg, mp['W3'],
                                    mp['b3'], params['up_mlp'],
                                    ln_g, ln_b, w1h_msg, True)
        else:
            H, P = _update_tc(H, parts[0], parts[1], deg, mp['W3'],
                              mp['b3'], params['up_mlp'],
                              ln_g, ln_b, w1h_msg, False)

    return (H, gsum.reshape(D))


# speedup vs baseline: 1.0536x; 1.0101x over previous
import functools

import jax
import jax.numpy as jnp
from jax import lax
from jax.experimental import pallas as pl
from jax.experimental.pallas import tpu as pltpu
from jax.experimental.pallas import tpu_sc as plsc

N = 10000
D = 128
EDGE_DIM = 16
E = 320000
N_LAYERS = 3

NW = 32
CHUNK = 128
E8 = E // 8
GPW = 40960
E_PAD = 8 * GPW
HALF = E_PAD // 2
NC_H = HALF // NW // CHUNK
N_PAD = 10240

NB = 400
EB = 640
NI = GPW // EB


@functools.cache
def _sc_gather_kernel():
    mesh = plsc.VectorSubcoreMesh(core_axis_name="c", subcore_axis_name="s")

    @functools.partial(
        pl.kernel,
        mesh=mesh,
        out_type=jax.ShapeDtypeStruct((NW, NC_H, CHUNK, DP), jnp.float32),
        scratch_types=[
            pltpu.VMEM((NC_H, CHUNK), jnp.int32),
            pltpu.VMEM((CHUNK, DP), jnp.float32),
            pltpu.VMEM((CHUNK, DP), jnp.float32),
            pltpu.VMEM_SHARED((N, DP), jnp.float32),
            pltpu.SemaphoreType.DMA,
            pltpu.SemaphoreType.DMA,
        ],
    )
    def gather_k(table_hbm, idx_hbm, out_hbm, idx_v, buf0, buf1, tbl,
                 sem0, sem1):
        c = lax.axis_index("c")
        s = lax.axis_index("s")
        wid = s * 2 + c

        pltpu.sync_copy(table_hbm.at[pl.ds(s * 624, 624)],
                        tbl.at[pl.ds(s * 624, 624)])

        @pl.when(s == 0)
        def _():
            pltpu.sync_copy(table_hbm.at[pl.ds(9984, 16)],
                            tbl.at[pl.ds(9984, 16)])

        pltpu.sync_copy(idx_hbm.at[wid], idx_v)
        plsc.subcore_barrier()

        def body(i, carry):
            j0 = 2 * i
            j1 = j0 + 1
            c0 = pltpu.async_copy(tbl.at[idx_v.at[j0]], buf0, sem0)
            c1 = pltpu.async_copy(tbl.at[idx_v.at[j1]], buf1, sem1)
            c0.wait()
            pltpu.sync_copy(buf0, out_hbm.at[wid, j0])
            c1.wait()
            pltpu.sync_copy(buf1, out_hbm.at[wid, j1])
            return carry

        lax.fori_loop(0, NC_H // 2, body, 0)

    return gather_k


def _sc_gather(table, idx_r):
    return _sc_gather_kernel()(table, idx_r)


@functools.cache
def _sc_scatter_kernel():
    mesh = plsc.VectorSubcoreMesh(core_axis_name="c", subcore_axis_name="s")

    @functools.partial(
        pl.kernel,
        mesh=mesh,
        out_type=jax.ShapeDtypeStruct((2, N_PAD, D), jnp.float32),
        scratch_types=[
            pltpu.VMEM((NC_H, CHUNK), jnp.int32),
            pltpu.VMEM((CHUNK, D), jnp.float32),
            pltpu.VMEM((CHUNK, D), jnp.float32),
            pltpu.VMEM_SHARED((N_PAD, D), jnp.float32),
            pltpu.SemaphoreType.DMA,
            pltpu.SemaphoreType.DMA,
        ],
    )
    def scatter_k(m_hbm, idx_hbm, z_hbm, out_hbm,
                  idx_v, buf0, buf1, acc, sem0, sem1):
        c = lax.axis_index("c")
        s = lax.axis_index("s")
        wid = s * 2 + c
        rows_per_sub = N_PAD // 16

        pltpu.sync_copy(z_hbm, buf0)

        def zbody(t, carry):
            pltpu.sync_copy(
                buf0, acc.at[pl.ds(s * rows_per_sub + t * CHUNK, CHUNK)])
            return carry

        lax.fori_loop(0, rows_per_sub // CHUNK, zbody, 0)
        pltpu.sync_copy(idx_hbm.at[wid], idx_v)
        plsc.subcore_barrier()

        def body(i, carry):
            j0 = 2 * i
            j1 = j0 + 1
            c0 = pltpu.async_copy(m_hbm.at[wid, j0], buf0, sem0)
            c1 = pltpu.async_copy(m_hbm.at[wid, j1], buf1, sem1)
            c0.wait()
            pltpu.sync_copy(buf0, acc.at[idx_v.at[j0]], add=True)
            c1.wait()
            pltpu.sync_copy(buf1, acc.at[idx_v.at[j1]], add=True)
            return carry

        lax.fori_loop(0, NC_H // 2, body, 0)
        plsc.subcore_barrier()

        pltpu.sync_copy(acc.at[pl.ds(s * rows_per_sub, rows_per_sub)],
                        out_hbm.at[c, pl.ds(s * rows_per_sub, rows_per_sub)])

    return scatter_k


def _sc_scatter(m_r, idx_r, zeros_blk):
    return _sc_scatter_kernel()(m_r, idx_r, zeros_blk)


NCF = E_PAD // NW // CHUNK


@functools.cache
def _sc_deg_kernel():
    mesh = plsc.VectorSubcoreMesh(core_axis_name="c", subcore_axis_name="s")

    @functools.partial(
        pl.kernel,
        mesh=mesh,
        out_type=jax.ShapeDtypeStruct((2, N_PAD, D), jnp.float32),
        scratch_types=[
            pltpu.VMEM((NCF, CHUNK), jnp.int32),
            pltpu.VMEM((CHUNK, D), jnp.float32),
            pltpu.VMEM_SHARED((N_PAD, D), jnp.float32),
        ],
    )
    def deg_k(idx_hbm, z_hbm, ones_hbm, out_hbm, idx_v, buf0, acc):
        c = lax.axis_index("c")
        s = lax.axis_index("s")
        wid = s * 2 + c
        rows_per_sub = N_PAD // 16

        pltpu.sync_copy(z_hbm, buf0)

        def zbody(t, carry):
            pltpu.sync_copy(
                buf0, acc.at[pl.ds(s * rows_per_sub + t * CHUNK, CHUNK)])
            return carry

        lax.fori_loop(0, rows_per_sub // CHUNK, zbody, 0)
        pltpu.sync_copy(ones_hbm, buf0)
        pltpu.sync_copy(idx_hbm.at[wid], idx_v)
        plsc.subcore_barrier()

        def body(j, carry):
            pltpu.sync_copy(buf0, acc.at[idx_v.at[j]], add=True)
            return carry

        lax.fori_loop(0, NCF, body, 0)
        plsc.subcore_barrier()

        pltpu.sync_copy(acc.at[pl.ds(s * rows_per_sub, rows_per_sub)],
                        out_hbm.at[c, pl.ds(s * rows_per_sub, rows_per_sub)])

    return deg_k


def _sc_deg(idx_full, zeros_blk, ones_blk):
    return _sc_deg_kernel()(idx_full, zeros_blk, ones_blk)


def _full(shape):
    return pl.BlockSpec(shape, lambda i: (0,) * len(shape))


def _full2(shape):
    return pl.BlockSpec(shape, lambda i, c: (0,) * len(shape))


DP = D // 2


def _pack_bf16(x):
    lo = x[:, :DP].astype(jnp.bfloat16).astype(jnp.float32)
    hi = x[:, DP:].astype(jnp.bfloat16).astype(jnp.float32)
    lo_b = lax.bitcast_convert_type(lo, jnp.uint32)
    hi_b = lax.bitcast_convert_type(hi, jnp.uint32)
    packed = lo_b | lax.shift_right_logical(hi_b, jnp.uint32(16))
    return lax.bitcast_convert_type(packed, jnp.float32)


def _unpack_bf16(x):
    u = lax.bitcast_convert_type(x, jnp.uint32)
    lo = lax.bitcast_convert_type(u & jnp.uint32(0xFFFF0000), jnp.float32)
    hi = lax.bitcast_convert_type(
        lax.shift_left(u, jnp.uint32(16)), jnp.float32)
    return jnp.concatenate([lo, hi], axis=1)


def _node_tc(x, mp, ln_g, ln_b, w1h_msg):
    def body(x_ref, w1, b1, w2, b2, w3, b3, g, b, wm, h_ref, p_ref):
        xv = jnp.nan_to_num(x_ref[...], nan=0.0, posinf=0.0, neginf=0.0)
        h = jnp.maximum(xv @ w1[...] + b1[...], 0.0)
        h = jnp.maximum(h @ w2[...] + b2[...], 0.0)
        h = h @ w3[...] + b3[...]
        mu = jnp.mean(h, axis=-1, keepdims=True)
        var = jnp.mean((h - mu) ** 2, axis=-1, keepdims=True)
        hn = (h - mu) * lax.rsqrt(var + 1e-5) * g[...] + b[...]
        h_ref[...] = hn
        p_ref[...] = _pack_bf16(hn @ wm[...])

    return pl.pallas_call(
        body,
        grid=(N // NB,),
        in_specs=[
            pl.BlockSpec((NB, D), lambda i: (i, 0)),
            _full((D, D)), _full((1, D)), _full((D, D)), _full((1, D)),
            _full((D, D)), _full((1, D)), _full((1, D)), _full((1, D)),
            _full((D, D)),
        ],
        out_specs=[
            pl.BlockSpec((NB, D), lambda i: (i, 0)),
            pl.BlockSpec((NB, DP), lambda i: (i, 0)),
        ],
        out_shape=[
            jax.ShapeDtypeStruct((N, D), jnp.float32),
            jax.ShapeDtypeStruct((N, DP), jnp.float32),
        ],
    )(x, mp['W1'], mp['b1'].reshape(1, D), mp['W2'], mp['b2'].reshape(1, D),
      mp['W3'], mp['b3'].reshape(1, D), ln_g.reshape(1, D), ln_b.reshape(1, D),
      w1h_msg)


def _msg_tc(g_arr, ea2p, wbig_h, b1, w2, b2):
    def body(g_ref, ea_ref, wb_ref, b1r, w2r, b2r, m_ref):
        c = pl.program_id(1)
        wc = wb_ref[c]
        g = _unpack_bf16(g_ref[...])
        h = jnp.maximum(g + ea_ref[...] @ wc + b1r[...], 0.0)
        m_ref[...] = jnp.maximum(h @ w2r[...] + b2r[...], 0.0)

    return pl.pallas_call(
        body,
        grid=(NI, 4),
        in_specs=[
            pl.BlockSpec((EB, DP), lambda i, c: (c * NI + i, 0)),
            pl.BlockSpec((EB, D), lambda i, c: (i, 0)),
            pl.BlockSpec((4, D, D), lambda i, c: (0, 0, 0)),
            _full2((1, D)), _full2((D, D)), _full2((1, D)),
        ],
        out_specs=pl.BlockSpec((EB, D), lambda i, c: (c * NI + i, 0)),
        out_shape=jax.ShapeDtypeStruct((HALF, D), jnp.float32),
    )(g_arr, ea2p, wbig_h, b1.reshape(1, D), w2, b2.reshape(1, D))


def _update_tc(h, part_a, part_b, deg, w3m, b3m, up, ln_g, ln_b, w1h_msg,
               compute_mean):
    nb = N // NB
    w1 = up['W1']

    def body(h_ref, pa0_ref, pa1_ref, pb0_ref, pb1_ref, d0_ref, d1_ref,
             w3r, b3r, w1h, w1a, b1, w2, b2, w3, b3, g, b, wm, *outs):
        agg2 = (pa0_ref[0] + pa1_ref[0]) + (pb0_ref[0] + pb1_ref[0])
        d = (d0_ref[0] + d1_ref[0])[:, 0:1]
        agg = agg2 @ w3r[...] + d * b3r[...]
        hv = h_ref[...]
        u = jnp.maximum(hv @ w1h[...] + agg @ w1a[...] + b1[...], 0.0)
        u = jnp.maximum(u @ w2[...] + b2[...], 0.0)
        u = u @ w3[...] + b3[...]
        hh = hv + u
        mu = jnp.mean(hh, axis=-1, keepdims=True)
        var = jnp.mean((hh - mu) ** 2, axis=-1, keepdims=True)
        hn = (hh - mu) * lax.rsqrt(var + 1e-5) * g[...] + b[...]
        outs[0][...] = hn
        outs[1][...] = _pack_bf16(hn @ wm[...])
        if compute_mean:
            i = pl.program_id(0)
            gacc = outs[2]

            @pl.when(i == 0)
            def _():
                gacc[...] = jnp.zeros_like(gacc)

            gacc[...] += jnp.sum(hn, axis=0, keepdims=True)

            @pl.when(i == nb - 1)
            def _():
                gacc[...] = gacc[...] * (1.0 / N)

    out_specs = [
        pl.BlockSpec((NB, D), lambda i: (i, 0)),
        pl.BlockSpec((NB, DP), lambda i: (i, 0)),
    ]
    out_shape = [
        jax.ShapeDtypeStruct((N, D), jnp.float32),
        jax.ShapeDtypeStruct((N, DP), jnp.float32),
    ]
    if compute_mean:
        out_specs.append(_full((1, D)))
        out_shape.append(jax.ShapeDtypeStruct((1, D), jnp.float32))

    return pl.pallas_call(
        body,
        grid=(nb,),
        in_specs=[
            pl.BlockSpec((NB, D), lambda i: (i, 0)),
            pl.BlockSpec((1, NB, D), lambda i: (0, i, 0)),
            pl.BlockSpec((1, NB, D), lambda i: (1, i, 0)),
            pl.BlockSpec((1, NB, D), lambda i: (0, i, 0)),
            pl.BlockSpec((1, NB, D), lambda i: (1, i, 0)),
            pl.BlockSpec((1, NB, D), lambda i: (0, i, 0)),
            pl.BlockSpec((1, NB, D), lambda i: (1, i, 0)),
            _full((D, D)), _full((1, D)),
            _full((D, D)), _full((D, D)), _full((1, D)), _full((D, D)),
            _full((1, D)), _full((D, D)), _full((1, D)), _full((1, D)),
            _full((1, D)), _full((D, D)),
        ],
        out_specs=out_specs,
        out_shape=out_shape,
    )(h, part_a, part_a, part_b, part_b, deg, deg, w3m, b3m.reshape(1, D),
      w1[:D], w1[D:],
      up['b1'].reshape(1, D), up['W2'],
      up['b2'].reshape(1, D), up['W3'], up['b3'].reshape(1, D),
      ln_g.reshape(1, D), ln_b.reshape(1, D), w1h_msg)


def kernel(node_x, edge_index, edge_attr, params):
    node_x = node_x.astype(jnp.float32)
    edge_attr = edge_attr.astype(jnp.float32)
    src = edge_index[0].astype(jnp.int32)
    dst = edge_index[1].astype(jnp.int32)

    srcg = jnp.pad(src.reshape(E8, 8).T, ((0, 0), (0, GPW - E8)))
    dstg = jnp.pad(dst.reshape(E8, 8).T, ((0, 0), (0, GPW - E8)),
                   constant_values=N)
    src_h = srcg.reshape(2, NW, NC_H, CHUNK)
    dst_h = dstg.reshape(2, NW, NC_H, CHUNK)
    ea2p = jnp.pad(edge_attr.reshape(E8, 8 * EDGE_DIM),
                   ((0, GPW - E8), (0, 0)))

    mp = params['msg_mlp']
    w1h_msg = mp['W1'][:D]
    w1e = mp['W1'][D:]
    wbig = jnp.zeros((8, D, D), jnp.float32)
    for c in range(8):
        wbig = wbig.at[c, 16 * c:16 * (c + 1), :].set(w1e)
    ln_g, ln_b = params['ln_g'], params['ln_b']

    H, P = _node_tc(node_x, params['node_mlp'], ln_g, ln_b, w1h_msg)
    zeros_blk = jnp.zeros((CHUNK, D), jnp.float32)
    ones_blk = jnp.ones((CHUNK, D), jnp.float32)
    deg = _sc_deg(dstg.reshape(NW, NCF, CHUNK), zeros_blk, ones_blk)

    gsum = None
    for layer in range(N_LAYERS):
        parts = []
        for half in range(2):
            G = _sc_gather(P, src_h[half]).reshape(HALF, DP)
            M = _msg_tc(G, ea2p, wbig[4 * half:4 * half + 4], mp['b1'],
                        mp['W2'], mp['b2'])
            parts.append(_sc_scatter(M.reshape(NW, NC_H, CHUNK, D),
                                     dst_h[half], zeros_blk))
        last = layer == N_LAYERS - 1
        if last:
            H, P, gsum = _update_tc(H, parts[0], parts[1], deg, mp['W3'],
                                    mp['b3'], params['up_mlp'],
                                    ln_g, ln_b, w1h_msg, True)
        else:
            H, P = _update_tc(H, parts[0], parts[1], deg, mp['W3'],
                              mp['b3'], params['up_mlp'],
                              ln_g, ln_b, w1h_msg, False)

    return (H, gsum.reshape(D))


# speedup vs baseline: 1.3124x; 1.2456x over previous
import functools

import jax
import jax.numpy as jnp
from jax import lax
from jax.experimental import pallas as pl
from jax.experimental.pallas import tpu as pltpu
from jax.experimental.pallas import tpu_sc as plsc

N = 10000
D = 128
EDGE_DIM = 16
E = 320000
N_LAYERS = 3

NW = 32
CHUNK = 128
E8 = E // 8
GPW = 40960
E_PAD = 8 * GPW
HALF = E_PAD // 2
NC_H = HALF // NW // CHUNK
N_PAD = 10240

NB = 400
EB = 1280
NI = GPW // EB


@functools.cache
def _sc_gather_kernel():
    mesh = plsc.VectorSubcoreMesh(core_axis_name="c", subcore_axis_name="s")

    @functools.partial(
        pl.kernel,
        mesh=mesh,
        out_type=jax.ShapeDtypeStruct((NW, NC_H, CHUNK, D), jnp.float32),
        scratch_types=[
            pltpu.VMEM((NC_H, CHUNK), jnp.int32),
            pltpu.VMEM((CHUNK, D), jnp.float32),
            pltpu.VMEM((CHUNK, D), jnp.float32),
            pltpu.VMEM_SHARED((N, D), jnp.float32),
            pltpu.SemaphoreType.DMA,
            pltpu.SemaphoreType.DMA,
        ],
    )
    def gather_k(table_hbm, idx_hbm, out_hbm, idx_v, buf0, buf1, tbl,
                 sem0, sem1):
        c = lax.axis_index("c")
        s = lax.axis_index("s")
        wid = s * 2 + c

        pltpu.sync_copy(table_hbm.at[pl.ds(s * 624, 624)],
                        tbl.at[pl.ds(s * 624, 624)])

        @pl.when(s == 0)
        def _():
            pltpu.sync_copy(table_hbm.at[pl.ds(9984, 16)],
                            tbl.at[pl.ds(9984, 16)])

        pltpu.sync_copy(idx_hbm.at[wid], idx_v)
        plsc.subcore_barrier()

        def body(i, carry):
            j0 = 2 * i
            j1 = j0 + 1
            c0 = pltpu.async_copy(tbl.at[idx_v.at[j0]], buf0, sem0)
            c1 = pltpu.async_copy(tbl.at[idx_v.at[j1]], buf1, sem1)
            c0.wait()
            pltpu.sync_copy(buf0, out_hbm.at[wid, j0])
            c1.wait()
            pltpu.sync_copy(buf1, out_hbm.at[wid, j1])
            return carry

        lax.fori_loop(0, NC_H // 2, body, 0)

    return gather_k


def _sc_gather(table, idx_r):
    return _sc_gather_kernel()(table, idx_r)


@functools.cache
def _sc_scatter_kernel():
    mesh = plsc.VectorSubcoreMesh(core_axis_name="c", subcore_axis_name="s")

    @functools.partial(
        pl.kernel,
        mesh=mesh,
        out_type=jax.ShapeDtypeStruct((2, N_PAD, D), jnp.float32),
        scratch_types=[
            pltpu.VMEM((NC_H, CHUNK), jnp.int32),
            pltpu.VMEM((CHUNK, D), jnp.float32),
            pltpu.VMEM((CHUNK, D), jnp.float32),
            pltpu.VMEM_SHARED((N_PAD, D), jnp.float32),
            pltpu.SemaphoreType.DMA,
            pltpu.SemaphoreType.DMA,
        ],
    )
    def scatter_k(m_hbm, idx_hbm, z_hbm, out_hbm,
                  idx_v, buf0, buf1, acc, sem0, sem1):
        c = lax.axis_index("c")
        s = lax.axis_index("s")
        wid = s * 2 + c
        rows_per_sub = N_PAD // 16

        pltpu.sync_copy(z_hbm, buf0)

        def zbody(t, carry):
            pltpu.sync_copy(
                buf0, acc.at[pl.ds(s * rows_per_sub + t * CHUNK, CHUNK)])
            return carry

        lax.fori_loop(0, rows_per_sub // CHUNK, zbody, 0)
        pltpu.sync_copy(idx_hbm.at[wid], idx_v)
        plsc.subcore_barrier()

        def body(i, carry):
            j0 = 2 * i
            j1 = j0 + 1
            c0 = pltpu.async_copy(m_hbm.at[wid, j0], buf0, sem0)
            c1 = pltpu.async_copy(m_hbm.at[wid, j1], buf1, sem1)
            c0.wait()
            pltpu.sync_copy(buf0, acc.at[idx_v.at[j0]], add=True)
            c1.wait()
            pltpu.sync_copy(buf1, acc.at[idx_v.at[j1]], add=True)
            return carry

        lax.fori_loop(0, NC_H // 2, body, 0)
        plsc.subcore_barrier()

        pltpu.sync_copy(acc.at[pl.ds(s * rows_per_sub, rows_per_sub)],
                        out_hbm.at[c, pl.ds(s * rows_per_sub, rows_per_sub)])

    return scatter_k


def _sc_scatter(m_r, idx_r, zeros_blk):
    return _sc_scatter_kernel()(m_r, idx_r, zeros_blk)


NCF = E_PAD // NW // CHUNK


@functools.cache
def _sc_deg_kernel():
    mesh = plsc.VectorSubcoreMesh(core_axis_name="c", subcore_axis_name="s")

    @functools.partial(
        pl.kernel,
        mesh=mesh,
        out_type=jax.ShapeDtypeStruct((2, N_PAD, D), jnp.float32),
        scratch_types=[
            pltpu.VMEM((NCF, CHUNK), jnp.int32),
            pltpu.VMEM((CHUNK, D), jnp.float32),
            pltpu.VMEM_SHARED((N_PAD, D), jnp.float32),
        ],
    )
    def deg_k(idx_hbm, z_hbm, ones_hbm, out_hbm, idx_v, buf0, acc):
        c = lax.axis_index("c")
        s = lax.axis_index("s")
        wid = s * 2 + c
        rows_per_sub = N_PAD // 16

        pltpu.sync_copy(z_hbm, buf0)

        def zbody(t, carry):
            pltpu.sync_copy(
                buf0, acc.at[pl.ds(s * rows_per_sub + t * CHUNK, CHUNK)])
            return carry

        lax.fori_loop(0, rows_per_sub // CHUNK, zbody, 0)
        pltpu.sync_copy(ones_hbm, buf0)
        pltpu.sync_copy(idx_hbm.at[wid], idx_v)
        plsc.subcore_barrier()

        def body(j, carry):
            pltpu.sync_copy(buf0, acc.at[idx_v.at[j]], add=True)
            return carry

        lax.fori_loop(0, NCF, body, 0)
        plsc.subcore_barrier()

        pltpu.sync_copy(acc.at[pl.ds(s * rows_per_sub, rows_per_sub)],
                        out_hbm.at[c, pl.ds(s * rows_per_sub, rows_per_sub)])

    return deg_k


def _sc_deg(idx_full, zeros_blk, ones_blk):
    return _sc_deg_kernel()(idx_full, zeros_blk, ones_blk)


def _full(shape):
    return pl.BlockSpec(shape, lambda i: (0,) * len(shape))


def _full2(shape):
    return pl.BlockSpec(shape, lambda i, c: (0,) * len(shape))


def _node_tc(x, mp, ln_g, ln_b, w1h_msg):
    def body(x_ref, w1, b1, w2, b2, w3, b3, g, b, wm, h_ref, p_ref):
        xv = jnp.nan_to_num(x_ref[...], nan=0.0, posinf=0.0, neginf=0.0)
        h = jnp.maximum(xv @ w1[...] + b1[...], 0.0)
        h = jnp.maximum(h @ w2[...] + b2[...], 0.0)
        h = h @ w3[...] + b3[...]
        mu = jnp.mean(h, axis=-1, keepdims=True)
        var = jnp.mean((h - mu) ** 2, axis=-1, keepdims=True)
        hn = (h - mu) * lax.rsqrt(var + 1e-5) * g[...] + b[...]
        h_ref[...] = hn
        p_ref[...] = hn @ wm[...]

    return pl.pallas_call(
        body,
        grid=(N // NB,),
        in_specs=[
            pl.BlockSpec((NB, D), lambda i: (i, 0)),
            _full((D, D)), _full((1, D)), _full((D, D)), _full((1, D)),
            _full((D, D)), _full((1, D)), _full((1, D)), _full((1, D)),
            _full((D, D)),
        ],
        out_specs=[
            pl.BlockSpec((NB, D), lambda i: (i, 0)),
            pl.BlockSpec((NB, D), lambda i: (i, 0)),
        ],
        out_shape=[
            jax.ShapeDtypeStruct((N, D), jnp.float32),
            jax.ShapeDtypeStruct((N, D), jnp.float32),
        ],
    )(x, mp['W1'], mp['b1'].reshape(1, D), mp['W2'], mp['b2'].reshape(1, D),
      mp['W3'], mp['b3'].reshape(1, D), ln_g.reshape(1, D), ln_b.reshape(1, D),
      w1h_msg)


def _msg_tc(g_arr, ea2p, wbig_h, b1, w2, b2):
    def body(g_ref, ea_ref, wb_ref, b1r, w2r, b2r, m_ref):
        c = pl.program_id(1)
        wc = wb_ref[c]
        h = jnp.maximum(g_ref[...] + ea_ref[...] @ wc + b1r[...], 0.0)
        m_ref[...] = jnp.maximum(h @ w2r[...] + b2r[...], 0.0)

    return pl.pallas_call(
        body,
        grid=(NI, 4),
        in_specs=[
            pl.BlockSpec((EB, D), lambda i, c: (c * NI + i, 0)),
            pl.BlockSpec((EB, D), lambda i, c: (i, 0)),
            pl.BlockSpec((4, D, D), lambda i, c: (0, 0, 0)),
            _full2((1, D)), _full2((D, D)), _full2((1, D)),
        ],
        out_specs=pl.BlockSpec((EB, D), lambda i, c: (c * NI + i, 0)),
        out_shape=jax.ShapeDtypeStruct((HALF, D), jnp.float32),
    )(g_arr, ea2p, wbig_h, b1.reshape(1, D), w2, b2.reshape(1, D))


def _update_tc(h, part_a, part_b, deg, w3m, b3m, up, ln_g, ln_b, w1h_msg,
               compute_mean):
    nb = N // NB
    w1 = up['W1']

    def body(h_ref, pa0_ref, pa1_ref, pb0_ref, pb1_ref, d0_ref, d1_ref,
             w3r, b3r, w1h, w1a, b1, w2, b2, w3, b3, g, b, wm, *outs):
        agg2 = (pa0_ref[0] + pa1_ref[0]) + (pb0_ref[0] + pb1_ref[0])
        d = (d0_ref[0] + d1_ref[0])[:, 0:1]
        agg = agg2 @ w3r[...] + d * b3r[...]
        hv = h_ref[...]
        u = jnp.maximum(hv @ w1h[...] + agg @ w1a[...] + b1[...], 0.0)
        u = jnp.maximum(u @ w2[...] + b2[...], 0.0)
        u = u @ w3[...] + b3[...]
        hh = hv + u
        mu = jnp.mean(hh, axis=-1, keepdims=True)
        var = jnp.mean((hh - mu) ** 2, axis=-1, keepdims=True)
        hn = (hh - mu) * lax.rsqrt(var + 1e-5) * g[...] + b[...]
        outs[0][...] = hn
        outs[1][...] = hn @ wm[...]
        if compute_mean:
            i = pl.program_id(0)
            gacc = outs[2]

            @pl.when(i == 0)
            def _():
                gacc[...] = jnp.zeros_like(gacc)

            gacc[...] += jnp.sum(hn, axis=0, keepdims=True)

            @pl.when(i == nb - 1)
            def _():
                gacc[...] = gacc[...] * (1.0 / N)

    out_specs = [
        pl.BlockSpec((NB, D), lambda i: (i, 0)),
        pl.BlockSpec((NB, D), lambda i: (i, 0)),
    ]
    out_shape = [
        jax.ShapeDtypeStruct((N, D), jnp.float32),
        jax.ShapeDtypeStruct((N, D), jnp.float32),
    ]
    if compute_mean:
        out_specs.append(_full((1, D)))
        out_shape.append(jax.ShapeDtypeStruct((1, D), jnp.float32))

    return pl.pallas_call(
        body,
        grid=(nb,),
        in_specs=[
            pl.BlockSpec((NB, D), lambda i: (i, 0)),
            pl.BlockSpec((1, NB, D), lambda i: (0, i, 0)),
            pl.BlockSpec((1, NB, D), lambda i: (1, i, 0)),
            pl.BlockSpec((1, NB, D), lambda i: (0, i, 0)),
            pl.BlockSpec((1, NB, D), lambda i: (1, i, 0)),
            pl.BlockSpec((1, NB, D), lambda i: (0, i, 0)),
            pl.BlockSpec((1, NB, D), lambda i: (1, i, 0)),
            _full((D, D)), _full((1, D)),
            _full((D, D)), _full((D, D)), _full((1, D)), _full((D, D)),
            _full((1, D)), _full((D, D)), _full((1, D)), _full((1, D)),
            _full((1, D)), _full((D, D)),
        ],
        out_specs=out_specs,
        out_shape=out_shape,
    )(h, part_a, part_a, part_b, part_b, deg, deg, w3m, b3m.reshape(1, D),
      w1[:D], w1[D:],
      up['b1'].reshape(1, D), up['W2'],
      up['b2'].reshape(1, D), up['W3'], up['b3'].reshape(1, D),
      ln_g.reshape(1, D), ln_b.reshape(1, D), w1h_msg)


def kernel(node_x, edge_index, edge_attr, params):
    node_x = node_x.astype(jnp.float32)
    edge_attr = edge_attr.astype(jnp.float32)
    src = edge_index[0].astype(jnp.int32)
    dst = edge_index[1].astype(jnp.int32)

    srcg = jnp.pad(src.reshape(E8, 8).T, ((0, 0), (0, GPW - E8)))
    dstg = jnp.pad(dst.reshape(E8, 8).T, ((0, 0), (0, GPW - E8)),
                   constant_values=N)
    src_h = srcg.reshape(2, NW, NC_H, CHUNK)
    dst_h = dstg.reshape(2, NW, NC_H, CHUNK)
    ea2p = jnp.pad(edge_attr.reshape(E8, 8 * EDGE_DIM),
                   ((0, GPW - E8), (0, 0)))

    mp = params['msg_mlp']
    w1h_msg = mp['W1'][:D]
    w1e = mp['W1'][D:]
    wbig = jnp.zeros((8, D, D), jnp.float32)
    for c in range(8):
        wbig = wbig.at[c, 16 * c:16 * (c + 1), :].set(w1e)
    ln_g, ln_b = params['ln_g'], params['ln_b']

    H, P = _node_tc(node_x, params['node_mlp'], ln_g, ln_b, w1h_msg)
    zeros_blk = jnp.zeros((CHUNK, D), jnp.float32)
    ones_blk = jnp.ones((CHUNK, D), jnp.float32)
    deg = _sc_deg(dstg.reshape(NW, NCF, CHUNK), zeros_blk, ones_blk)

    gsum = None
    for layer in range(N_LAYERS):
        parts = []
        for half in range(2):
            G = _sc_gather(P, src_h[half]).reshape(HALF, D)
            M = _msg_tc(G, ea2p, wbig[4 * half:4 * half + 4], mp['b1'],
                        mp['W2'], mp['b2'])
            parts.append(_sc_scatter(M.reshape(NW, NC_H, CHUNK, D),
                                     dst_h[half], zeros_blk))
        last = layer == N_LAYERS - 1
        if last:
            H, P, gsum = _update_tc(H, parts[0], parts[1], deg, mp['W3'],
                                    mp['b3'], params['up_mlp'],
                                    ln_g, ln_b, w1h_msg, True)
        else:
            H, P = _update_tc(H, parts[0], parts[1], deg, mp['W3'],
                              mp['b3'], params['up_mlp'],
                              ln_g, ln_b, w1h_msg, False)

    return (H, gsum.reshape(D))


# speedup vs baseline: 1.5131x; 1.1529x over previous
import functools

import jax
import jax.numpy as jnp
from jax import lax
from jax.experimental import pallas as pl
from jax.experimental.pallas import tpu as pltpu
from jax.experimental.pallas import tpu_sc as plsc

N = 10000
D = 128
EDGE_DIM = 16
E = 320000
N_LAYERS = 3

NW = 32
CHUNK = 128
E8 = E // 8
GPW = 40960
E_PAD = 8 * GPW
HALF = E_PAD // 2
NC_H = HALF // NW // CHUNK
N_PAD = 10240

NB = 400
EB = 2560
NI = GPW // EB


@functools.cache
def _sc_gather_kernel():
    mesh = plsc.VectorSubcoreMesh(core_axis_name="c", subcore_axis_name="s")

    @functools.partial(
        pl.kernel,
        mesh=mesh,
        out_type=jax.ShapeDtypeStruct((NW, NC_H, CHUNK, D), jnp.float32),
        scratch_types=[
            pltpu.VMEM((NC_H, CHUNK), jnp.int32),
            pltpu.VMEM((CHUNK, D), jnp.float32),
            pltpu.VMEM((CHUNK, D), jnp.float32),
            pltpu.VMEM_SHARED((N, D), jnp.float32),
            pltpu.SemaphoreType.DMA,
            pltpu.SemaphoreType.DMA,
        ],
    )
    def gather_k(table_hbm, idx_hbm, out_hbm, idx_v, buf0, buf1, tbl,
                 sem0, sem1):
        c = lax.axis_index("c")
        s = lax.axis_index("s")
        wid = s * 2 + c

        pltpu.sync_copy(table_hbm.at[pl.ds(s * 624, 624)],
                        tbl.at[pl.ds(s * 624, 624)])

        @pl.when(s == 0)
        def _():
            pltpu.sync_copy(table_hbm.at[pl.ds(9984, 16)],
                            tbl.at[pl.ds(9984, 16)])

        pltpu.sync_copy(idx_hbm.at[wid], idx_v)
        plsc.subcore_barrier()

        def body(i, carry):
            j0 = 2 * i
            j1 = j0 + 1
            c0 = pltpu.async_copy(tbl.at[idx_v.at[j0]], buf0, sem0)
            c1 = pltpu.async_copy(tbl.at[idx_v.at[j1]], buf1, sem1)
            c0.wait()
            pltpu.sync_copy(buf0, out_hbm.at[wid, j0])
            c1.wait()
            pltpu.sync_copy(buf1, out_hbm.at[wid, j1])
            return carry

        lax.fori_loop(0, NC_H // 2, body, 0)

    return gather_k


def _sc_gather(table, idx_r):
    return _sc_gather_kernel()(table, idx_r)


@functools.cache
def _sc_scatter_kernel():
    mesh = plsc.VectorSubcoreMesh(core_axis_name="c", subcore_axis_name="s")

    @functools.partial(
        pl.kernel,
        mesh=mesh,
        out_type=jax.ShapeDtypeStruct((2, N_PAD, D), jnp.float32),
        scratch_types=[
            pltpu.VMEM((NC_H, CHUNK), jnp.int32),
            pltpu.VMEM((CHUNK, D), jnp.float32),
            pltpu.VMEM((CHUNK, D), jnp.float32),
            pltpu.VMEM_SHARED((N_PAD, D), jnp.float32),
            pltpu.SemaphoreType.DMA,
            pltpu.SemaphoreType.DMA,
        ],
    )
    def scatter_k(m_hbm, idx_hbm, z_hbm, out_hbm,
                  idx_v, buf0, buf1, acc, sem0, sem1):
        c = lax.axis_index("c")
        s = lax.axis_index("s")
        wid = s * 2 + c
        rows_per_sub = N_PAD // 16

        pltpu.sync_copy(z_hbm, buf0)

        def zbody(t, carry):
            pltpu.sync_copy(
                buf0, acc.at[pl.ds(s * rows_per_sub + t * CHUNK, CHUNK)])
            return carry

        lax.fori_loop(0, rows_per_sub // CHUNK, zbody, 0)
        pltpu.sync_copy(idx_hbm.at[wid], idx_v)
        plsc.subcore_barrier()

        def body(i, carry):
            j0 = 2 * i
            j1 = j0 + 1
            c0 = pltpu.async_copy(m_hbm.at[wid, j0], buf0, sem0)
            c1 = pltpu.async_copy(m_hbm.at[wid, j1], buf1, sem1)
            c0.wait()
            pltpu.sync_copy(buf0, acc.at[idx_v.at[j0]], add=True)
            c1.wait()
            pltpu.sync_copy(buf1, acc.at[idx_v.at[j1]], add=True)
            return carry

        lax.fori_loop(0, NC_H // 2, body, 0)
        plsc.subcore_barrier()

        pltpu.sync_copy(acc.at[pl.ds(s * rows_per_sub, rows_per_sub)],
                        out_hbm.at[c, pl.ds(s * rows_per_sub, rows_per_sub)])

    return scatter_k


def _sc_scatter(m_r, idx_r, zeros_blk):
    return _sc_scatter_kernel()(m_r, idx_r, zeros_blk)


NCF = E_PAD // NW // CHUNK


@functools.cache
def _sc_deg_kernel():
    mesh = plsc.VectorSubcoreMesh(core_axis_name="c", subcore_axis_name="s")

    @functools.partial(
        pl.kernel,
        mesh=mesh,
        out_type=jax.ShapeDtypeStruct((2, N_PAD, D), jnp.float32),
        scratch_types=[
            pltpu.VMEM((NCF, CHUNK), jnp.int32),
            pltpu.VMEM((CHUNK, D), jnp.float32),
            pltpu.VMEM_SHARED((N_PAD, D), jnp.float32),
        ],
    )
    def deg_k(idx_hbm, z_hbm, ones_hbm, out_hbm, idx_v, buf0, acc):
        c = lax.axis_index("c")
        s = lax.axis_index("s")
        wid = s * 2 + c
        rows_per_sub = N_PAD // 16

        pltpu.sync_copy(z_hbm, buf0)

        def zbody(t, carry):
            pltpu.sync_copy(
                buf0, acc.at[pl.ds(s * rows_per_sub + t * CHUNK, CHUNK)])
            return carry

        lax.fori_loop(0, rows_per_sub // CHUNK, zbody, 0)
        pltpu.sync_copy(ones_hbm, buf0)
        pltpu.sync_copy(idx_hbm.at[wid], idx_v)
        plsc.subcore_barrier()

        def body(j, carry):
            pltpu.sync_copy(buf0, acc.at[idx_v.at[j]], add=True)
            return carry

        lax.fori_loop(0, NCF, body, 0)
        plsc.subcore_barrier()

        pltpu.sync_copy(acc.at[pl.ds(s * rows_per_sub, rows_per_sub)],
                        out_hbm.at[c, pl.ds(s * rows_per_sub, rows_per_sub)])

    return deg_k


def _sc_deg(idx_full, zeros_blk, ones_blk):
    return _sc_deg_kernel()(idx_full, zeros_blk, ones_blk)


def _full(shape):
    return pl.BlockSpec(shape, lambda i: (0,) * len(shape))


def _full2(shape):
    return pl.BlockSpec(shape, lambda i, c: (0,) * len(shape))


def _node_tc(x, mp, ln_g, ln_b, w1h_msg):
    def body(x_ref, w1, b1, w2, b2, w3, b3, g, b, wm, h_ref, p_ref):
        xv = jnp.nan_to_num(x_ref[...], nan=0.0, posinf=0.0, neginf=0.0)
        h = jnp.maximum(xv @ w1[...] + b1[...], 0.0)
        h = jnp.maximum(h @ w2[...] + b2[...], 0.0)
        h = h @ w3[...] + b3[...]
        mu = jnp.mean(h, axis=-1, keepdims=True)
        var = jnp.mean((h - mu) ** 2, axis=-1, keepdims=True)
        hn = (h - mu) * lax.rsqrt(var + 1e-5) * g[...] + b[...]
        h_ref[...] = hn
        p_ref[...] = hn @ wm[...]

    return pl.pallas_call(
        body,
        grid=(N // NB,),
        in_specs=[
            pl.BlockSpec((NB, D), lambda i: (i, 0)),
            _full((D, D)), _full((1, D)), _full((D, D)), _full((1, D)),
            _full((D, D)), _full((1, D)), _full((1, D)), _full((1, D)),
            _full((D, D)),
        ],
        out_specs=[
            pl.BlockSpec((NB, D), lambda i: (i, 0)),
            pl.BlockSpec((NB, D), lambda i: (i, 0)),
        ],
        out_shape=[
            jax.ShapeDtypeStruct((N, D), jnp.float32),
            jax.ShapeDtypeStruct((N, D), jnp.float32),
        ],
    )(x, mp['W1'], mp['b1'].reshape(1, D), mp['W2'], mp['b2'].reshape(1, D),
      mp['W3'], mp['b3'].reshape(1, D), ln_g.reshape(1, D), ln_b.reshape(1, D),
      w1h_msg)


def _msg_tc(g_arr, ea2p, wbig_h, b1, w2, b2):
    def body(g_ref, ea_ref, wb_ref, b1r, w2r, b2r, m_ref):
        c = pl.program_id(1)
        wc = wb_ref[c]
        h = jnp.maximum(g_ref[...] + ea_ref[...] @ wc + b1r[...], 0.0)
        m_ref[...] = jnp.maximum(h @ w2r[...] + b2r[...], 0.0)

    return pl.pallas_call(
        body,
        grid=(NI, 4),
        in_specs=[
            pl.BlockSpec((EB, D), lambda i, c: (c * NI + i, 0)),
            pl.BlockSpec((EB, D), lambda i, c: (i, 0)),
            pl.BlockSpec((4, D, D), lambda i, c: (0, 0, 0)),
            _full2((1, D)), _full2((D, D)), _full2((1, D)),
        ],
        out_specs=pl.BlockSpec((EB, D), lambda i, c: (c * NI + i, 0)),
        out_shape=jax.ShapeDtypeStruct((HALF, D), jnp.float32),
    )(g_arr, ea2p, wbig_h, b1.reshape(1, D), w2, b2.reshape(1, D))


def _update_tc(h, part_a, part_b, deg, w3m, b3m, up, ln_g, ln_b, w1h_msg,
               compute_mean):
    nb = N // NB
    w1 = up['W1']

    def body(h_ref, pa0_ref, pa1_ref, pb0_ref, pb1_ref, d0_ref, d1_ref,
             w3r, b3r, w1h, w1a, b1, w2, b2, w3, b3, g, b, wm, *outs):
        agg2 = (pa0_ref[0] + pa1_ref[0]) + (pb0_ref[0] + pb1_ref[0])
        d = (d0_ref[0] + d1_ref[0])[:, 0:1]
        agg = agg2 @ w3r[...] + d * b3r[...]
        hv = h_ref[...]
        u = jnp.maximum(hv @ w1h[...] + agg @ w1a[...] + b1[...], 0.0)
        u = jnp.maximum(u @ w2[...] + b2[...], 0.0)
        u = u @ w3[...] + b3[...]
        hh = hv + u
        mu = jnp.mean(hh, axis=-1, keepdims=True)
        var = jnp.mean((hh - mu) ** 2, axis=-1, keepdims=True)
        hn = (hh - mu) * lax.rsqrt(var + 1e-5) * g[...] + b[...]
        outs[0][...] = hn
        outs[1][...] = hn @ wm[...]
        if compute_mean:
            i = pl.program_id(0)
            gacc = outs[2]

            @pl.when(i == 0)
            def _():
                gacc[...] = jnp.zeros_like(gacc)

            gacc[...] += jnp.sum(hn, axis=0, keepdims=True)

            @pl.when(i == nb - 1)
            def _():
                gacc[...] = gacc[...] * (1.0 / N)

    out_specs = [
        pl.BlockSpec((NB, D), lambda i: (i, 0)),
        pl.BlockSpec((NB, D), lambda i: (i, 0)),
    ]
    out_shape = [
        jax.ShapeDtypeStruct((N, D), jnp.float32),
        jax.ShapeDtypeStruct((N, D), jnp.float32),
    ]
    if compute_mean:
        out_specs.append(_full((1, D)))
        out_shape.append(jax.ShapeDtypeStruct((1, D), jnp.float32))

    return pl.pallas_call(
        body,
        grid=(nb,),
        in_specs=[
            pl.BlockSpec((NB, D), lambda i: (i, 0)),
            pl.BlockSpec((1, NB, D), lambda i: (0, i, 0)),
            pl.BlockSpec((1, NB, D), lambda i: (1, i, 0)),
            pl.BlockSpec((1, NB, D), lambda i: (0, i, 0)),
            pl.BlockSpec((1, NB, D), lambda i: (1, i, 0)),
            pl.BlockSpec((1, NB, D), lambda i: (0, i, 0)),
            pl.BlockSpec((1, NB, D), lambda i: (1, i, 0)),
            _full((D, D)), _full((1, D)),
            _full((D, D)), _full((D, D)), _full((1, D)), _full((D, D)),
            _full((1, D)), _full((D, D)), _full((1, D)), _full((1, D)),
            _full((1, D)), _full((D, D)),
        ],
        out_specs=out_specs,
        out_shape=out_shape,
    )(h, part_a, part_a, part_b, part_b, deg, deg, w3m, b3m.reshape(1, D),
      w1[:D], w1[D:],
      up['b1'].reshape(1, D), up['W2'],
      up['b2'].reshape(1, D), up['W3'], up['b3'].reshape(1, D),
      ln_g.reshape(1, D), ln_b.reshape(1, D), w1h_msg)


def kernel(node_x, edge_index, edge_attr, params):
    node_x = node_x.astype(jnp.float32)
    edge_attr = edge_attr.astype(jnp.float32)
    src = edge_index[0].astype(jnp.int32)
    dst = edge_index[1].astype(jnp.int32)

    srcg = jnp.pad(src.reshape(E8, 8).T, ((0, 0), (0, GPW - E8)))
    dstg = jnp.pad(dst.reshape(E8, 8).T, ((0, 0), (0, GPW - E8)),
                   constant_values=N)
    src_h = srcg.reshape(2, NW, NC_H, CHUNK)
    dst_h = dstg.reshape(2, NW, NC_H, CHUNK)
    ea2p = jnp.pad(edge_attr.reshape(E8, 8 * EDGE_DIM),
                   ((0, GPW - E8), (0, 0)))

    mp = params['msg_mlp']
    w1h_msg = mp['W1'][:D]
    w1e = mp['W1'][D:]
    wbig = jnp.zeros((8, D, D), jnp.float32)
    for c in range(8):
        wbig = wbig.at[c, 16 * c:16 * (c + 1), :].set(w1e)
    ln_g, ln_b = params['ln_g'], params['ln_b']

    H, P = _node_tc(node_x, params['node_mlp'], ln_g, ln_b, w1h_msg)
    zeros_blk = jnp.zeros((CHUNK, D), jnp.float32)
    ones_blk = jnp.ones((CHUNK, D), jnp.float32)
    deg = _sc_deg(dstg.reshape(NW, NCF, CHUNK), zeros_blk, ones_blk)

    gsum = None
    for layer in range(N_LAYERS):
        parts = []
        for half in range(2):
            G = _sc_gather(P, src_h[half]).reshape(HALF, D)
            M = _msg_tc(G, ea2p, wbig[4 * half:4 * half + 4], mp['b1'],
                        mp['W2'], mp['b2'])
            parts.append(_sc_scatter(M.reshape(NW, NC_H, CHUNK, D),
                                     dst_h[half], zeros_blk))
        last = layer == N_LAYERS - 1
        if last:
            H, P, gsum = _update_tc(H, parts[0], parts[1], deg, mp['W3'],
                                    mp['b3'], params['up_mlp'],
                                    ln_g, ln_b, w1h_msg, True)
        else:
            H, P = _update_tc(H, parts[0], parts[1], deg, mp['W3'],
                              mp['b3'], params['up_mlp'],
                              ln_g, ln_b, w1h_msg, False)

    return (H, gsum.reshape(D))
